# baseline (speedup 1.0000x reference)
import functools

import jax
import jax.numpy as jnp
from jax import lax
from jax.experimental import pallas as pl
from jax.experimental.pallas import tpu as pltpu
from jax.experimental.pallas import tpu_sc as plsc

B = 16
MAX_LEN = 2048
D = 1024
HALF = MAX_LEN // 2

NC = 2
NS = 16

_BITS = [1 << k for k in range(10, -1, -1)]

_mesh = plsc.VectorSubcoreMesh(core_axis_name="c", subcore_axis_name="s")


@functools.partial(
    pl.kernel,
    mesh=_mesh,
    out_type=[
        jax.ShapeDtypeStruct((B * MAX_LEN, D), jnp.float32),
        jax.ShapeDtypeStruct((B, MAX_LEN), jnp.int32),
    ],
    scratch_types=[
        pltpu.VMEM((32,), jnp.int32),
        pltpu.VMEM((HALF,), jnp.int32),
        pltpu.SemaphoreType.DMA,
    ],
    compiler_params=pltpu.CompilerParams(use_tc_tiling_on_sc=False,
                                         needs_layout_passes=False),
)
def _pack(cu_hbm, flat_hbm, zeros_hbm, padded_hbm, mask_hbm,
          cu_v, mask_v, sem):
    wid = lax.axis_index("s") * NC + lax.axis_index("c")
    b = wid // 2
    p0 = (wid % 2) * HALF

    pltpu.sync_copy(cu_hbm, cu_v)
    lane = lax.iota(jnp.int32, 16)
    sel = lane == b
    start_b = jnp.sum(jnp.where(sel, cu_v[pl.ds(0, 16)], 0))
    len_b = jnp.sum(jnp.where(sel, cu_v[pl.ds(16, 16)], 0))

    n_real = jnp.clip(len_b - p0, 0, HALF)
    n_pad = HALF - n_real
    src0 = start_b + p0
    out0 = b * MAX_LEN + p0

    for s in _BITS:
        off = n_real & ~(2 * s - 1)

        @pl.when((n_real & s) != 0)
        def _issue_copy(s=s, off=off):
            pltpu.make_async_copy(flat_hbm.at[pl.ds(src0 + off, s)],
                                  padded_hbm.at[pl.ds(out0 + off, s)],
                                  sem).start()

        zoff = n_pad & ~(2 * s - 1)

        @pl.when((n_pad & s) != 0)
        def _issue_zero(s=s, zoff=zoff):
            pltpu.make_async_copy(zeros_hbm.at[pl.ds(0, s)],
                                  padded_hbm.at[pl.ds(out0 + n_real + zoff, s)],
                                  sem).start()

    def mrow(k, carry):
        mask_v[pl.ds(k * 16, 16)] = (lane + (p0 + k * 16) < len_b).astype(
            jnp.int32)
        return carry

    lax.fori_loop(0, HALF // 16, mrow, 0)
    pltpu.sync_copy(mask_v, mask_hbm.at[b, pl.ds(p0, HALF)])

    for s in _BITS:
        off = n_real & ~(2 * s - 1)

        @pl.when((n_real & s) != 0)
        def _wait_copy(s=s, off=off):
            pltpu.make_async_copy(flat_hbm.at[pl.ds(src0 + off, s)],
                                  padded_hbm.at[pl.ds(out0 + off, s)],
                                  sem).wait()

        zoff = n_pad & ~(2 * s - 1)

        @pl.when((n_pad & s) != 0)
        def _wait_zero(s=s, zoff=zoff):
            pltpu.make_async_copy(zeros_hbm.at[pl.ds(0, s)],
                                  padded_hbm.at[pl.ds(out0 + n_real + zoff, s)],
                                  sem).wait()


def kernel(flat, cu_seqlens):
    cu32 = jnp.concatenate([cu_seqlens[:B],
                            cu_seqlens[1:] - cu_seqlens[:-1]])
    zeros = jnp.zeros((HALF, D), jnp.float32)
    padded_flat, mask = _pack(cu32, flat, zeros)
    return padded_flat.reshape(B, MAX_LEN, D), mask


# speedup vs baseline: 14.8499x; 14.8499x over previous
import functools

import jax
import jax.numpy as jnp
from jax import lax
from jax.experimental import pallas as pl
from jax.experimental.pallas import tpu as pltpu
from jax.experimental.pallas import tpu_sc as plsc

B = 16
MAX_LEN = 2048
D = 1024
HALF = MAX_LEN // 2

NC = 2
NS = 16

C = 32
NBUF = 3
NCH_MAX = HALF // C
ZB = 16
NZ_MAX = HALF // ZB
Z_OUT = 8

_mesh = plsc.VectorSubcoreMesh(core_axis_name="c", subcore_axis_name="s")


@functools.partial(
    pl.kernel,
    mesh=_mesh,
    out_type=[
        jax.ShapeDtypeStruct((B * MAX_LEN, D), jnp.float32),
        jax.ShapeDtypeStruct((B, MAX_LEN), jnp.int32),
    ],
    scratch_types=(
        [pltpu.VMEM((32,), jnp.int32),
         pltpu.VMEM((HALF,), jnp.int32),
         pltpu.VMEM((ZB, D), jnp.float32)]
        + [pltpu.VMEM((C, D), jnp.float32) for _ in range(NBUF)]
        + [pltpu.SemaphoreType.DMA for _ in range(2 * NBUF + 2)]
    ),
    compiler_params=pltpu.CompilerParams(use_tc_tiling_on_sc=False,
                                         needs_layout_passes=False),
)
def _pack(cu_hbm, flat_hbm, zeros_hbm, padded_hbm, mask_hbm,
          cu_v, mask_v, zero_v, *bufs_and_sems):
    bufs = bufs_and_sems[:NBUF]
    insem = bufs_and_sems[NBUF:2 * NBUF]
    outsem = bufs_and_sems[2 * NBUF:3 * NBUF]
    zsem = bufs_and_sems[3 * NBUF]
    msem = bufs_and_sems[3 * NBUF + 1]

    wid = lax.axis_index("s") * NC + lax.axis_index("c")
    b = wid // 2
    p0 = (wid % 2) * HALF

    pltpu.sync_copy(cu_hbm, cu_v)
    lane = lax.iota(jnp.int32, 16)
    sel = lane == b
    start_b = jnp.sum(jnp.where(sel, cu_v[pl.ds(0, 16)], 0))
    len_b = jnp.sum(jnp.where(sel, cu_v[pl.ds(16, 16)], 0))

    n_real = jnp.clip(len_b - p0, 0, HALF)
    n_pad = HALF - n_real
    src0 = start_b + p0
    out0 = b * MAX_LEN + p0

    pltpu.sync_copy(zeros_hbm, zero_v)

    nzf = n_pad // ZB
    zbase = out0 + n_real

    def zissue(j, carry):
        pltpu.make_async_copy(zero_v, padded_hbm.at[pl.ds(zbase + j * ZB, ZB)],
                              zsem).start()

        @pl.when(j >= Z_OUT)
        def _():
            pltpu.make_async_copy(
                zero_v, padded_hbm.at[pl.ds(zbase, ZB)], zsem).wait()

        return carry

    lax.fori_loop(0, nzf, zissue, 0)

    zrem_base = zbase + nzf * ZB
    zoff = 0
    for s in [ZB >> k for k in range(1, ZB.bit_length())]:
        @pl.when((n_pad & s) != 0)
        def _zbit(s=s, zoff=zoff):
            pltpu.make_async_copy(zero_v.at[pl.ds(0, s)],
                                  padded_hbm.at[pl.ds(zrem_base + zoff, s)],
                                  zsem).start()

        zoff = zoff + jnp.where((n_pad & s) != 0, s, 0)

    nch = jnp.where(n_real >= C, (n_real + C - 1) // C, 0)

    def off(i):
        return jnp.minimum(i * C, n_real - C)

    def gather(i, slot):
        pltpu.make_async_copy(flat_hbm.at[pl.ds(src0 + off(i), C)],
                              bufs[slot], insem[slot]).start()

    for j in range(NBUF):
        @pl.when(j < nch)
        def _prime(j=j):
            gather(j, j)

    for i in range(NCH_MAX):
        slot = i % NBUF

        @pl.when(i < nch)
        def _chunk(i=i, slot=slot):
            pltpu.make_async_copy(flat_hbm.at[pl.ds(src0 + off(i), C)],
                                  bufs[slot], insem[slot]).wait()
            pltpu.make_async_copy(bufs[slot],
                                  padded_hbm.at[pl.ds(out0 + off(i), C)],
                                  outsem[slot]).start()

        @pl.when(i + NBUF < nch)
        def _next(i=i, slot=slot):
            pltpu.make_async_copy(bufs[slot],
                                  padded_hbm.at[pl.ds(out0, C)],
                                  outsem[slot]).wait()
            gather(i + NBUF, slot)

    for s in range(NBUF):
        @pl.when(nch > s)
        def _drain(s=s):
            pltpu.make_async_copy(bufs[s], padded_hbm.at[pl.ds(out0, C)],
                                  outsem[s]).wait()

    roff = 0
    for s in [C >> k for k in range(1, C.bit_length())]:
        @pl.when(jnp.logical_and(n_real < C, (n_real & s) != 0))
        def _rbit(s=s, roff=roff):
            pltpu.sync_copy(flat_hbm.at[pl.ds(src0 + roff, s)],
                            bufs[0].at[pl.ds(0, s)])
            pltpu.sync_copy(bufs[0].at[pl.ds(0, s)],
                            padded_hbm.at[pl.ds(out0 + roff, s)])

        roff = roff + jnp.where((n_real & s) != 0, s, 0)

    def mrow(k, carry):
        mask_v[pl.ds(k * 16, 16)] = (lane + (p0 + k * 16) < len_b).astype(
            jnp.int32)
        return carry

    lax.fori_loop(0, HALF // 16, mrow, 0)
    pltpu.make_async_copy(mask_v, mask_hbm.at[b, pl.ds(p0, HALF)],
                          msem).start()

    def zdrain(j, carry):
        pltpu.make_async_copy(zero_v, padded_hbm.at[pl.ds(zbase, ZB)],
                              zsem).wait()
        return carry

    lax.fori_loop(0, jnp.minimum(nzf, Z_OUT), zdrain, 0)
    for s in [ZB >> k for k in range(1, ZB.bit_length())]:
        @pl.when((n_pad & s) != 0)
        def _zbitw(s=s):
            pltpu.make_async_copy(zero_v.at[pl.ds(0, s)],
                                  padded_hbm.at[pl.ds(zbase, s)],
                                  zsem).wait()

    pltpu.make_async_copy(mask_v, mask_hbm.at[b, pl.ds(p0, HALF)],
                          msem).wait()


def kernel(flat, cu_seqlens):
    cu32 = jnp.concatenate([cu_seqlens[:B],
                            cu_seqlens[1:] - cu_seqlens[:-1]])
    zeros = jnp.zeros((ZB, D), jnp.float32)
    padded_flat, mask = _pack(cu32, flat, zeros)
    return padded_flat.reshape(B, MAX_LEN, D), mask


# speedup vs baseline: 35.6121x; 2.3981x over previous
import functools

import jax
import jax.numpy as jnp
from jax import lax
from jax.experimental import pallas as pl
from jax.experimental.pallas import tpu as pltpu
from jax.experimental.pallas import tpu_sc as plsc

B = 16
MAX_LEN = 2048
D = 1024
HALF = MAX_LEN // 2

NC = 2
NS = 16

C = 32
NBUF = 2
NCH = HALF // C
NPBUF = 4

_mesh = plsc.VectorSubcoreMesh(core_axis_name="c", subcore_axis_name="s")


@functools.partial(
    pl.kernel,
    mesh=_mesh,
    out_type=[
        jax.ShapeDtypeStruct((B * MAX_LEN, D), jnp.float32),
        jax.ShapeDtypeStruct((B, MAX_LEN), jnp.int32),
    ],
    scratch_types=(
        [pltpu.VMEM((32,), jnp.int32),
         pltpu.VMEM((C, D), jnp.float32),
         pltpu.VMEM((8, MAX_LEN), jnp.int32),
         pltpu.VMEM((32,), jnp.int32)]
        + [pltpu.VMEM((C, D), jnp.float32) for _ in range(NBUF)]
        + [pltpu.VMEM((32,), jnp.int32) for _ in range(NBUF)]
        + [pltpu.VMEM((32,), jnp.int32) for _ in range(NPBUF)]
        + [pltpu.SemaphoreType.DMA for _ in range(2 * NBUF + NPBUF)]
    ),
    compiler_params=pltpu.CompilerParams(needs_layout_passes=False),
)
def _pack(cu_hbm, flat_hbm, zeros_hbm, padded_hbm, mask_hbm,
          cu_v, zero_v, mbuf, sidx, *rest):
    bufs = rest[:NBUF]
    gidx = rest[NBUF:2 * NBUF]
    pidx = rest[2 * NBUF:2 * NBUF + NPBUF]
    insem = rest[2 * NBUF + NPBUF:3 * NBUF + NPBUF]
    outsem = rest[3 * NBUF + NPBUF:4 * NBUF + NPBUF]
    psem = rest[4 * NBUF + NPBUF:]

    wid = lax.axis_index("s") * NC + lax.axis_index("c")
    b = wid // 2
    p0 = (wid % 2) * HALF

    pltpu.sync_copy(cu_hbm, cu_v)
    lane = lax.iota(jnp.int32, 16)
    sel = lane == b
    start_b = jnp.sum(jnp.where(sel, cu_v[pl.ds(0, 16)], 0))
    len_b = jnp.sum(jnp.where(sel, cu_v[pl.ds(16, 16)], 0))

    n_real = jnp.clip(len_b - p0, 0, HALF)
    n_pad = HALF - n_real
    src0 = start_b + p0
    out0 = b * MAX_LEN + p0
    zbase = out0 + n_real

    pltpu.sync_copy(zeros_hbm, zero_v)

    nzch = (n_pad + C - 1) // C

    for j in range(NCH):
        slot = j % NPBUF

        @pl.when(j < nzch)
        def _pad(j=j, slot=slot):
            if j >= NPBUF:
                pltpu.make_async_copy(zero_v, padded_hbm.at[pidx[slot]],
                                      psem[slot]).wait()
            for h in (0, 16):
                q = jnp.minimum(j * C + h + lane, n_pad - 1)
                pidx[slot][pl.ds(h, 16)] = zbase + q
            pltpu.make_async_copy(zero_v, padded_hbm.at[pidx[slot]],
                                  psem[slot]).start()

    nch = (n_real + C - 1) // C

    def fill_gidx(i, slot):
        for h in (0, 16):
            q = jnp.minimum(i * C + h + lane, n_real - 1)
            gidx[slot][pl.ds(h, 16)] = src0 + q

    for j in range(NBUF):
        @pl.when(j < nch)
        def _prime(j=j):
            fill_gidx(j, j)
            pltpu.make_async_copy(flat_hbm.at[gidx[j]], bufs[j],
                                  insem[j]).start()

    for i in range(NCH):
        slot = i % NBUF

        @pl.when(i < nch)
        def _chunk(i=i, slot=slot):
            pltpu.make_async_copy(flat_hbm.at[gidx[slot]], bufs[slot],
                                  insem[slot]).wait()

            @pl.when((i + 1) * C <= n_real)
            def _full():
                pltpu.make_async_copy(bufs[slot],
                                      padded_hbm.at[pl.ds(out0 + i * C, C)],
                                      outsem[slot]).start()

            @pl.when((i + 1) * C > n_real)
            def _boundary():
                for h in (0, 16):
                    q = jnp.minimum(i * C + h + lane, n_real - 1)
                    sidx[pl.ds(h, 16)] = out0 + q
                pltpu.make_async_copy(bufs[slot], padded_hbm.at[sidx],
                                      outsem[slot]).start()

        @pl.when(i + NBUF < nch)
        def _next(i=i, slot=slot):
            pltpu.make_async_copy(bufs[slot],
                                  padded_hbm.at[pl.ds(out0, C)],
                                  outsem[slot]).wait()
            fill_gidx(i + NBUF, slot)
            pltpu.make_async_copy(flat_hbm.at[gidx[slot]], bufs[slot],
                                  insem[slot]).start()

    @pl.when(wid < 2)
    def _mask():
        def mrow(r, carry):
            len_r = jnp.sum(jnp.where(lane == wid * 8 + r,
                                      cu_v[pl.ds(16, 16)], 0))

            def mcol(k, carry2):
                mbuf[r, pl.ds(k * 16, 16)] = (lane + k * 16 < len_r).astype(
                    jnp.int32)
                return carry2

            lax.fori_loop(0, MAX_LEN // 16, mcol, 0)
            return carry

        lax.fori_loop(0, 8, mrow, 0)
        pltpu.sync_copy(mbuf, mask_hbm.at[pl.ds(wid * 8, 8)])

    for s in range(NBUF):
        @pl.when(nch > s)
        def _drain(s=s):
            pltpu.make_async_copy(bufs[s], padded_hbm.at[pl.ds(out0, C)],
                                  outsem[s]).wait()

    for s in range(NPBUF):
        @pl.when(nzch > s)
        def _pdrain(s=s):
            pltpu.make_async_copy(zero_v, padded_hbm.at[pidx[s]],
                                  psem[s]).wait()


def kernel(flat, cu_seqlens):
    cu32 = jnp.concatenate([cu_seqlens[:B],
                            cu_seqlens[1:] - cu_seqlens[:-1]])
    zeros = jnp.zeros((C, D), jnp.float32)
    padded_flat, mask = _pack(cu32, flat, zeros)
    return padded_flat.reshape(B, MAX_LEN, D), mask
